# baseline (speedup 1.0000x reference)
import functools

import jax
import jax.numpy as jnp
from jax.experimental import pallas as pl

VOCAB = 1000
T = 50
EMB = 32
BATCH = 1024
BB = 32


def _fused_kernel(x_ref, tok_ref, pos_ref, w_ref, b_ref, out_ref):
    idx = x_ref[...][:, :, None]
    iota = jax.lax.broadcasted_iota(jnp.int32, (BB, T, VOCAB), 2)
    onehot = (idx == iota).astype(jnp.float32).reshape(BB * T, VOCAB)
    emb = jnp.dot(onehot, tok_ref[...], preferred_element_type=jnp.float32)
    h = emb.reshape(BB, T, EMB) + pos_ref[...][None, :, :]
    logits = (
        jnp.dot(h.reshape(BB * T, EMB), w_ref[...],
                preferred_element_type=jnp.float32)
        + b_ref[...]
    )
    out_ref[...] = logits.reshape(BB, T, VOCAB)


@functools.partial(jax.jit, static_argnames=())
def kernel(x, tok_table, pos_table, W, b):
    grid = (BATCH // BB,)
    return pl.pallas_call(
        _fused_kernel,
        grid=grid,
        in_specs=[
            pl.BlockSpec((BB, T), lambda i: (i, 0)),
            pl.BlockSpec((VOCAB, EMB), lambda i: (0, 0)),
            pl.BlockSpec((T, EMB), lambda i: (0, 0)),
            pl.BlockSpec((EMB, VOCAB), lambda i: (0, 0)),
            pl.BlockSpec((1, VOCAB), lambda i: (0, 0)),
        ],
        out_specs=pl.BlockSpec((BB, T, VOCAB), lambda i: (i, 0, 0)),
        out_shape=jax.ShapeDtypeStruct((BATCH, T, VOCAB), jnp.float32),
    )(x, tok_table, pos_table, W, b.reshape(1, VOCAB))
